# Initial kernel scaffold; baseline (speedup 1.0000x reference)
import jax
import jax.numpy as jnp
from jax.experimental import pallas as pl


def kernel(features, adj, labels_indices, labels_values, W1, W2, W3):
    raise NotImplementedError("write your pallas kernel here")


# speedup vs baseline: 2.4012x; 2.4012x over previous
import functools

import jax
import jax.numpy as jnp
from jax import lax
from jax.experimental import pallas as pl
from jax.experimental.pallas import tpu as pltpu
from jax.experimental.pallas import tpu_sc as plsc

N = 4096
F_IN = 256
H1 = 128
H2 = 64
NNZ = 65536

BM = 512
DB = 512


def _matmul_kernel(x_ref, w_ref, o_ref):
    o_ref[...] = lax.dot_general(
        x_ref[...], w_ref[...], (((1,), (0,)), ((), ())),
        preferred_element_type=jnp.float32)


def _spmm_relu_kernel(a_ref, b_ref, o_ref):
    acc = lax.dot_general(
        a_ref[...], b_ref[...], (((1,), (0,)), ((), ())),
        preferred_element_type=jnp.float32)
    o_ref[...] = jnp.maximum(acc, 0.0)


def _z_kernel(a_ref, b_ref, eps_ref, zm_ref, z_ref, kl_ref, klacc):
    i = pl.program_id(0)
    zc = lax.dot_general(
        a_ref[...], b_ref[...], (((1,), (0,)), ((), ())),
        preferred_element_type=jnp.float32)
    zm = zc[:, :H2]
    zs = zc[:, H2:]
    ez = jnp.exp(zs)
    zm_ref[...] = zm
    z_ref[...] = zm + eps_ref[...] * ez
    term = 1.0 + 2.0 * zs - zm * zm - ez * ez

    @pl.when(i == 0)
    def _():
        klacc[...] = term

    @pl.when(i > 0)
    def _():
        klacc[...] += term

    @pl.when(i == pl.num_programs(0) - 1)
    def _():
        kl_ref[...] = jnp.sum(klacc[...], keepdims=True)


def _decode_kernel(zi_ref, zf_ref, s1_ref, cge_ref, accs, accc):
    i = pl.program_id(0)
    j = pl.program_id(1)
    first = jnp.logical_and(i == 0, j == 0)
    zj = zf_ref[pl.ds(j * DB, DB), :]
    x = lax.dot_general(
        zi_ref[...], zj, (((1,), (1,)), ((), ())),
        preferred_element_type=jnp.float32)
    sp = jnp.maximum(x, 0.0) + jnp.log1p(jnp.exp(-jnp.abs(x)))
    ge = (x >= 0.0).astype(jnp.float32)

    @pl.when(first)
    def _():
        accs[...] = sp
        accc[...] = ge

    @pl.when(jnp.logical_not(first))
    def _():
        accs[...] += sp
        accc[...] += ge

    @pl.when(jnp.logical_and(i == pl.num_programs(0) - 1,
                             j == pl.num_programs(1) - 1))
    def _():
        s1_ref[...] = jnp.sum(accs[...], keepdims=True)
        cge_ref[...] = jnp.sum(accc[...], keepdims=True)


def _dense_chain(features, adj, eps, W1, W23):
    xw1 = pl.pallas_call(
        _matmul_kernel,
        grid=(N // BM,),
        in_specs=[pl.BlockSpec((BM, F_IN), lambda i: (i, 0)),
                  pl.BlockSpec((F_IN, H1), lambda i: (0, 0))],
        out_specs=pl.BlockSpec((BM, H1), lambda i: (i, 0)),
        out_shape=jax.ShapeDtypeStruct((N, H1), jnp.float32),
    )(features, W1)

    h1 = pl.pallas_call(
        _spmm_relu_kernel,
        grid=(N // BM,),
        in_specs=[pl.BlockSpec((BM, N), lambda i: (i, 0)),
                  pl.BlockSpec((N, H1), lambda i: (0, 0))],
        out_specs=pl.BlockSpec((BM, H1), lambda i: (i, 0)),
        out_shape=jax.ShapeDtypeStruct((N, H1), jnp.float32),
    )(adj, xw1)

    hw23 = pl.pallas_call(
        _matmul_kernel,
        grid=(N // BM,),
        in_specs=[pl.BlockSpec((BM, H1), lambda i: (i, 0)),
                  pl.BlockSpec((H1, 2 * H2), lambda i: (0, 0))],
        out_specs=pl.BlockSpec((BM, 2 * H2), lambda i: (i, 0)),
        out_shape=jax.ShapeDtypeStruct((N, 2 * H2), jnp.float32),
    )(h1, W23)

    z_mean, z, klsum = pl.pallas_call(
        _z_kernel,
        grid=(N // BM,),
        in_specs=[pl.BlockSpec((BM, N), lambda i: (i, 0)),
                  pl.BlockSpec((N, 2 * H2), lambda i: (0, 0)),
                  pl.BlockSpec((BM, H2), lambda i: (i, 0))],
        out_specs=[pl.BlockSpec((BM, H2), lambda i: (i, 0)),
                   pl.BlockSpec((BM, H2), lambda i: (i, 0)),
                   pl.BlockSpec((1, 1), lambda i: (0, 0))],
        out_shape=[jax.ShapeDtypeStruct((N, H2), jnp.float32),
                   jax.ShapeDtypeStruct((N, H2), jnp.float32),
                   jax.ShapeDtypeStruct((1, 1), jnp.float32)],
        scratch_shapes=[pltpu.VMEM((BM, H2), jnp.float32)],
    )(adj, hw23, eps)
    return z_mean, z, klsum


def _decode(z):
    s1, cge = pl.pallas_call(
        _decode_kernel,
        grid=(N // DB, N // DB),
        in_specs=[pl.BlockSpec((DB, H2), lambda i, j: (i, 0)),
                  pl.BlockSpec((N, H2), lambda i, j: (0, 0))],
        out_specs=[pl.BlockSpec((1, 1), lambda i, j: (0, 0)),
                   pl.BlockSpec((1, 1), lambda i, j: (0, 0))],
        out_shape=[jax.ShapeDtypeStruct((1, 1), jnp.float32),
                   jax.ShapeDtypeStruct((1, 1), jnp.float32)],
        scratch_shapes=[pltpu.VMEM((DB, DB), jnp.float32),
                        pltpu.VMEM((DB, DB), jnp.float32)],
    )(z, z)
    return s1[0, 0], cge[0, 0]


_E_PER_TILE = NNZ // 32
_CHUNK = 256
_GROUPS = _CHUNK // 16


def _sc_sparse_body(z_hbm, ii_hbm, jj_hbm, vv_hbm, out_hbm,
                    ii_v, jj_v, vv_v, zi_v, zj_v, q_v, o_v, sem0, sem1):
    c = lax.axis_index("c")
    s = lax.axis_index("s")
    wid = s * 2 + c
    base = wid * _E_PER_TILE
    pltpu.sync_copy(ii_hbm.at[pl.ds(base, _E_PER_TILE)], ii_v)
    pltpu.sync_copy(jj_hbm.at[pl.ds(base, _E_PER_TILE)], jj_v)
    pltpu.sync_copy(vv_hbm.at[pl.ds(base, _E_PER_TILE)], vv_v)

    lanes = lax.iota(jnp.int32, 16)
    zero16 = jnp.zeros((16,), jnp.float32)

    def chunk_body(ck, carry):
        acc_s2, acc_corr = carry
        off = ck * _CHUNK
        pltpu.async_copy(z_hbm.at[ii_v.at[pl.ds(off, _CHUNK)]], zi_v, sem0).wait()
        pltpu.async_copy(z_hbm.at[jj_v.at[pl.ds(off, _CHUNK)]], zj_v, sem1).wait()

        def row_body(r, _):
            p0 = zi_v[r, pl.ds(0, 16)] * zj_v[r, pl.ds(0, 16)]
            p1 = zi_v[r, pl.ds(16, 16)] * zj_v[r, pl.ds(16, 16)]
            p2 = zi_v[r, pl.ds(32, 16)] * zj_v[r, pl.ds(32, 16)]
            p3 = zi_v[r, pl.ds(48, 16)] * zj_v[r, pl.ds(48, 16)]
            q_v[pl.ds(r * 16, 16)] = (p0 + p1) + (p2 + p3)
            return 0

        lax.fori_loop(0, _CHUNK, row_body, 0)

        def group_body(g, carry2):
            a_s2, a_corr = carry2
            flat0 = (g * 16 + lanes) * 16
            x = zero16
            for l in range(16):
                x = x + plsc.load_gather(q_v, [flat0 + l])
            vals = vv_v[pl.ds(off + g * 16, 16)]
            a_s2 = a_s2 + vals * x
            a_corr = a_corr + jnp.where(x >= 0.0, 1.0, -1.0)
            return (a_s2, a_corr)

        return lax.fori_loop(0, _GROUPS, group_body, (acc_s2, acc_corr))

    acc_s2, acc_corr = lax.fori_loop(
        0, _E_PER_TILE // _CHUNK, chunk_body, (zero16, zero16))
    o_v[0, :] = acc_s2
    o_v[1, :] = acc_corr
    pltpu.sync_copy(o_v, out_hbm.at[wid])


def _sc_sparse(z, idx_i, idx_j, values):
    mesh = plsc.VectorSubcoreMesh(core_axis_name="c", subcore_axis_name="s")
    run = pl.kernel(
        _sc_sparse_body,
        out_type=jax.ShapeDtypeStruct((32, 2, 16), jnp.float32),
        mesh=mesh,
        compiler_params=pltpu.CompilerParams(needs_layout_passes=False,
                                             use_tc_tiling_on_sc=False),
        scratch_types=[
            pltpu.VMEM((_E_PER_TILE,), jnp.int32),
            pltpu.VMEM((_E_PER_TILE,), jnp.int32),
            pltpu.VMEM((_E_PER_TILE,), jnp.float32),
            pltpu.VMEM((_CHUNK, H2), jnp.float32),
            pltpu.VMEM((_CHUNK, H2), jnp.float32),
            pltpu.VMEM((_CHUNK * 16,), jnp.float32),
            pltpu.VMEM((2, 16), jnp.float32),
            pltpu.SemaphoreType.DMA,
            pltpu.SemaphoreType.DMA,
        ],
    )
    return run(z, idx_i, idx_j, values)


def kernel(features, adj, labels_indices, labels_values, W1, W2, W3):
    W23 = jnp.concatenate([W2, W3], axis=1)
    eps = jax.random.normal(jax.random.key(42), (N, H2), dtype=jnp.float32)

    z_mean, z, klsum = _dense_chain(features, adj, eps, W1, W23)
    s1, cge = _decode(z)

    idx_i = labels_indices[:, 0].astype(jnp.int32)
    idx_j = labels_indices[:, 1].astype(jnp.int32)
    sc_out = _sc_sparse(z, idx_i, idx_j, labels_values)
    s2 = jnp.sum(sc_out[:, 0, :])
    corr = jnp.sum(sc_out[:, 1, :])

    n2 = jnp.float32(N * N)
    cost_pre = (s1 - s2) / n2
    kl = 0.5 * klsum[0, 0] / n2
    cost = cost_pre - kl
    accuracy = ((n2 - cge) + corr) / n2
    return (cost, accuracy, z_mean, cost_pre)


# speedup vs baseline: 2.7007x; 1.1247x over previous
import functools

import jax
import jax.numpy as jnp
from jax import lax
from jax.experimental import pallas as pl
from jax.experimental.pallas import tpu as pltpu
from jax.experimental.pallas import tpu_sc as plsc

N = 4096
F_IN = 256
H1 = 128
H2 = 64
NNZ = 65536

BM = 512
DB = 512


def _matmul_kernel(x_ref, w_ref, o_ref):
    o_ref[...] = lax.dot_general(
        x_ref[...], w_ref[...], (((1,), (0,)), ((), ())),
        preferred_element_type=jnp.float32)
def _gcn_kernel(a_ref, xw1_ref, w23_ref, eps_ref,
                zm_ref, z_ref, kl_ref,
                adjb, h1s, hw23b, klacc):
    i = pl.program_id(0)

    @pl.when(i < 8)
    def _():
        blk = a_ref[...]
        acc = lax.dot_general(
            blk, xw1_ref[...], (((1,), (0,)), ((), ())),
            preferred_element_type=jnp.float32)
        h1s[pl.ds(i * BM, BM), :] = jnp.maximum(acc, 0.0)
        adjb[pl.ds(i * BM, BM), :] = blk.astype(jnp.bfloat16)

    @pl.when(i == 8)
    def _():
        hw = lax.dot_general(
            h1s[...], w23_ref[...], (((1,), (0,)), ((), ())),
            preferred_element_type=jnp.float32)
        hw23b[...] = hw.astype(jnp.bfloat16)

    @pl.when(i >= 8)
    def _():
        r = jnp.maximum(i - 8, 0)
        zc = lax.dot_general(
            adjb[pl.ds(r * BM, BM), :], hw23b[...], (((1,), (0,)), ((), ())),
            preferred_element_type=jnp.float32)
        zm = zc[:, :H2]
        zs = zc[:, H2:]
        ez = jnp.exp(zs)
        zm_ref[...] = zm
        z_ref[...] = zm + eps_ref[...] * ez
        term = 1.0 + 2.0 * zs - zm * zm - ez * ez

        @pl.when(i == 8)
        def _():
            klacc[...] = term

        @pl.when(i > 8)
        def _():
            klacc[...] += term

        @pl.when(i == 15)
        def _():
            kl_ref[...] = jnp.sum(klacc[...], keepdims=True)


def _decode_kernel(zi_ref, zf_ref, s1_ref, cge_ref, accs, accc):
    i = pl.program_id(0)
    j = pl.program_id(1)
    first = jnp.logical_and(i == 0, j == 0)
    zj = zf_ref[pl.ds(j * DB, DB), :]
    x = lax.dot_general(
        zi_ref[...], zj, (((1,), (1,)), ((), ())),
        preferred_element_type=jnp.float32)
    sp = jnp.maximum(x, 0.0) + jnp.log1p(jnp.exp(-jnp.abs(x)))
    ge = (x >= 0.0).astype(jnp.float32)

    @pl.when(first)
    def _():
        accs[...] = sp
        accc[...] = ge

    @pl.when(jnp.logical_not(first))
    def _():
        accs[...] += sp
        accc[...] += ge

    @pl.when(jnp.logical_and(i == pl.num_programs(0) - 1,
                             j == pl.num_programs(1) - 1))
    def _():
        s1_ref[...] = jnp.sum(accs[...], keepdims=True)
        cge_ref[...] = jnp.sum(accc[...], keepdims=True)


def _dense_chain(features, adj, eps, W1, W23):
    xw1 = pl.pallas_call(
        _matmul_kernel,
        grid=(N // BM,),
        in_specs=[pl.BlockSpec((BM, F_IN), lambda i: (i, 0)),
                  pl.BlockSpec((F_IN, H1), lambda i: (0, 0))],
        out_specs=pl.BlockSpec((BM, H1), lambda i: (i, 0)),
        out_shape=jax.ShapeDtypeStruct((N, H1), jnp.float32),
    )(features, W1)

    z_mean, z, klsum = pl.pallas_call(
        _gcn_kernel,
        grid=(16,),
        in_specs=[pl.BlockSpec((BM, N), lambda i: (jnp.minimum(i, 7), 0)),
                  pl.BlockSpec((N, H1), lambda i: (0, 0)),
                  pl.BlockSpec((H1, 2 * H2), lambda i: (0, 0)),
                  pl.BlockSpec((BM, H2), lambda i: (jnp.maximum(i - 8, 0), 0))],
        out_specs=[pl.BlockSpec((BM, H2), lambda i: (jnp.maximum(i - 8, 0), 0)),
                   pl.BlockSpec((BM, H2), lambda i: (jnp.maximum(i - 8, 0), 0)),
                   pl.BlockSpec((1, 1), lambda i: (0, 0))],
        out_shape=[jax.ShapeDtypeStruct((N, H2), jnp.float32),
                   jax.ShapeDtypeStruct((N, H2), jnp.float32),
                   jax.ShapeDtypeStruct((1, 1), jnp.float32)],
        scratch_shapes=[pltpu.VMEM((N, N), jnp.bfloat16),
                        pltpu.VMEM((N, H1), jnp.float32),
                        pltpu.VMEM((N, 2 * H2), jnp.bfloat16),
                        pltpu.VMEM((BM, H2), jnp.float32)],
        compiler_params=pltpu.CompilerParams(
            vmem_limit_bytes=100 * 1024 * 1024),
    )(adj, xw1, W23, eps)
    return z_mean, z, klsum


def _decode(z):
    s1, cge = pl.pallas_call(
        _decode_kernel,
        grid=(N // DB, N // DB),
        in_specs=[pl.BlockSpec((DB, H2), lambda i, j: (i, 0)),
                  pl.BlockSpec((N, H2), lambda i, j: (0, 0))],
        out_specs=[pl.BlockSpec((1, 1), lambda i, j: (0, 0)),
                   pl.BlockSpec((1, 1), lambda i, j: (0, 0))],
        out_shape=[jax.ShapeDtypeStruct((1, 1), jnp.float32),
                   jax.ShapeDtypeStruct((1, 1), jnp.float32)],
        scratch_shapes=[pltpu.VMEM((DB, DB), jnp.float32),
                        pltpu.VMEM((DB, DB), jnp.float32)],
    )(z, z)
    return s1[0, 0], cge[0, 0]


_E_PER_TILE = NNZ // 32
_CHUNK = 256
_GROUPS = _CHUNK // 16


def _sc_sparse_body(z_hbm, ii_hbm, jj_hbm, vv_hbm, out_hbm,
                    ii_v, jj_v, vv_v, zi_v, zj_v, q_v, o_v, sem0, sem1):
    c = lax.axis_index("c")
    s = lax.axis_index("s")
    wid = s * 2 + c
    base = wid * _E_PER_TILE
    pltpu.sync_copy(ii_hbm.at[pl.ds(base, _E_PER_TILE)], ii_v)
    pltpu.sync_copy(jj_hbm.at[pl.ds(base, _E_PER_TILE)], jj_v)
    pltpu.sync_copy(vv_hbm.at[pl.ds(base, _E_PER_TILE)], vv_v)

    lanes = lax.iota(jnp.int32, 16)
    zero16 = jnp.zeros((16,), jnp.float32)

    def chunk_body(ck, carry):
        acc_s2, acc_corr = carry
        off = ck * _CHUNK
        pltpu.async_copy(z_hbm.at[ii_v.at[pl.ds(off, _CHUNK)]], zi_v, sem0).wait()
        pltpu.async_copy(z_hbm.at[jj_v.at[pl.ds(off, _CHUNK)]], zj_v, sem1).wait()

        def row_body(r, _):
            p0 = zi_v[r, pl.ds(0, 16)] * zj_v[r, pl.ds(0, 16)]
            p1 = zi_v[r, pl.ds(16, 16)] * zj_v[r, pl.ds(16, 16)]
            p2 = zi_v[r, pl.ds(32, 16)] * zj_v[r, pl.ds(32, 16)]
            p3 = zi_v[r, pl.ds(48, 16)] * zj_v[r, pl.ds(48, 16)]
            q_v[pl.ds(r * 16, 16)] = (p0 + p1) + (p2 + p3)
            return 0

        lax.fori_loop(0, _CHUNK, row_body, 0)

        def group_body(g, carry2):
            a_s2, a_corr = carry2
            flat0 = (g * 16 + lanes) * 16
            x = zero16
            for l in range(16):
                x = x + plsc.load_gather(q_v, [flat0 + l])
            vals = vv_v[pl.ds(off + g * 16, 16)]
            a_s2 = a_s2 + vals * x
            a_corr = a_corr + jnp.where(x >= 0.0, 1.0, -1.0)
            return (a_s2, a_corr)

        return lax.fori_loop(0, _GROUPS, group_body, (acc_s2, acc_corr))

    acc_s2, acc_corr = lax.fori_loop(
        0, _E_PER_TILE // _CHUNK, chunk_body, (zero16, zero16))
    o_v[0, :] = acc_s2
    o_v[1, :] = acc_corr
    pltpu.sync_copy(o_v, out_hbm.at[wid])


def _sc_sparse(z, idx_i, idx_j, values):
    mesh = plsc.VectorSubcoreMesh(core_axis_name="c", subcore_axis_name="s")
    run = pl.kernel(
        _sc_sparse_body,
        out_type=jax.ShapeDtypeStruct((32, 2, 16), jnp.float32),
        mesh=mesh,
        compiler_params=pltpu.CompilerParams(needs_layout_passes=False,
                                             use_tc_tiling_on_sc=False),
        scratch_types=[
            pltpu.VMEM((_E_PER_TILE,), jnp.int32),
            pltpu.VMEM((_E_PER_TILE,), jnp.int32),
            pltpu.VMEM((_E_PER_TILE,), jnp.float32),
            pltpu.VMEM((_CHUNK, H2), jnp.float32),
            pltpu.VMEM((_CHUNK, H2), jnp.float32),
            pltpu.VMEM((_CHUNK * 16,), jnp.float32),
            pltpu.VMEM((2, 16), jnp.float32),
            pltpu.SemaphoreType.DMA,
            pltpu.SemaphoreType.DMA,
        ],
    )
    return run(z, idx_i, idx_j, values)


def kernel(features, adj, labels_indices, labels_values, W1, W2, W3):
    W23 = jnp.concatenate([W2, W3], axis=1)
    eps = jax.random.normal(jax.random.key(42), (N, H2), dtype=jnp.float32)

    z_mean, z, klsum = _dense_chain(features, adj, eps, W1, W23)
    s1, cge = _decode(z)

    idx_i = labels_indices[:, 0].astype(jnp.int32)
    idx_j = labels_indices[:, 1].astype(jnp.int32)
    sc_out = _sc_sparse(z, idx_i, idx_j, labels_values)
    s2 = jnp.sum(sc_out[:, 0, :])
    corr = jnp.sum(sc_out[:, 1, :])

    n2 = jnp.float32(N * N)
    cost_pre = (s1 - s2) / n2
    kl = 0.5 * klsum[0, 0] / n2
    cost = cost_pre - kl
    accuracy = ((n2 - cge) + corr) / n2
    return (cost, accuracy, z_mean, cost_pre)


# speedup vs baseline: 2.8675x; 1.0618x over previous
import functools

import jax
import jax.numpy as jnp
from jax import lax
from jax.experimental import pallas as pl
from jax.experimental.pallas import tpu as pltpu
from jax.experimental.pallas import tpu_sc as plsc

N = 4096
F_IN = 256
H1 = 128
H2 = 64
NNZ = 65536

BM = 512
DB = 512


def _matmul_kernel(x_ref, w_ref, o_ref):
    o_ref[...] = lax.dot_general(
        x_ref[...], w_ref[...], (((1,), (0,)), ((), ())),
        preferred_element_type=jnp.float32)
def _gcn_kernel(a_ref, xw1_ref, w23_ref, eps_ref,
                zm_ref, z_ref, zb_ref, kl_ref,
                adjb, h1s, hw23b, klacc):
    i = pl.program_id(0)

    @pl.when(i < 8)
    def _():
        blk = a_ref[...]
        acc = lax.dot_general(
            blk, xw1_ref[...], (((1,), (0,)), ((), ())),
            preferred_element_type=jnp.float32)
        h1s[pl.ds(i * BM, BM), :] = jnp.maximum(acc, 0.0)
        adjb[pl.ds(i * BM, BM), :] = blk.astype(jnp.bfloat16)

    @pl.when(i == 8)
    def _():
        hw = lax.dot_general(
            h1s[...], w23_ref[...], (((1,), (0,)), ((), ())),
            preferred_element_type=jnp.float32)
        hw23b[...] = hw.astype(jnp.bfloat16)

    @pl.when(i >= 8)
    def _():
        r = jnp.maximum(i - 8, 0)
        zc = lax.dot_general(
            adjb[pl.ds(r * BM, BM), :], hw23b[...], (((1,), (0,)), ((), ())),
            preferred_element_type=jnp.float32)
        zm = zc[:, :H2]
        zs = zc[:, H2:]
        ez = jnp.exp(zs)
        zv = zm + eps_ref[...] * ez
        zm_ref[...] = zm
        z_ref[...] = zv
        zb_ref[...] = zv.astype(jnp.bfloat16)
        term = 1.0 + 2.0 * zs - zm * zm - ez * ez

        @pl.when(i == 8)
        def _():
            klacc[...] = term

        @pl.when(i > 8)
        def _():
            klacc[...] += term

        @pl.when(i == 15)
        def _():
            kl_ref[...] = jnp.sum(klacc[...], keepdims=True)


def _decode_kernel(zi_ref, zf_ref, s1_ref, cge_ref, accs, accc):
    i = pl.program_id(0)
    j = pl.program_id(1)
    first = jnp.logical_and(i == 0, j == 0)
    zj = zf_ref[pl.ds(j * DB, DB), :]
    x = lax.dot_general(
        zi_ref[...], zj, (((1,), (1,)), ((), ())),
        preferred_element_type=jnp.float32)
    sp = jnp.maximum(x, 0.0) + jnp.log1p(jnp.exp(-jnp.abs(x)))
    ge = (x >= 0.0).astype(jnp.float32)
    sp_part = jnp.sum(sp, axis=0, keepdims=True)
    ge_part = jnp.sum(ge, axis=0, keepdims=True)

    @pl.when(first)
    def _():
        accs[...] = sp_part
        accc[...] = ge_part

    @pl.when(jnp.logical_not(first))
    def _():
        accs[...] += sp_part
        accc[...] += ge_part

    @pl.when(jnp.logical_and(i == pl.num_programs(0) - 1,
                             j == pl.num_programs(1) - 1))
    def _():
        s1_ref[...] = jnp.sum(accs[...], keepdims=True)
        cge_ref[...] = jnp.sum(accc[...], keepdims=True)


def _dense_chain(features, adj, eps, W1, W23):
    xw1 = pl.pallas_call(
        _matmul_kernel,
        grid=(N // BM,),
        in_specs=[pl.BlockSpec((BM, F_IN), lambda i: (i, 0)),
                  pl.BlockSpec((F_IN, H1), lambda i: (0, 0))],
        out_specs=pl.BlockSpec((BM, H1), lambda i: (i, 0)),
        out_shape=jax.ShapeDtypeStruct((N, H1), jnp.float32),
    )(features, W1)

    z_mean, z, zb, klsum = pl.pallas_call(
        _gcn_kernel,
        grid=(16,),
        in_specs=[pl.BlockSpec((BM, N), lambda i: (jnp.minimum(i, 7), 0)),
                  pl.BlockSpec((N, H1), lambda i: (0, 0)),
                  pl.BlockSpec((H1, 2 * H2), lambda i: (0, 0)),
                  pl.BlockSpec((BM, H2), lambda i: (jnp.maximum(i - 8, 0), 0))],
        out_specs=[pl.BlockSpec((BM, H2), lambda i: (jnp.maximum(i - 8, 0), 0)),
                   pl.BlockSpec((BM, H2), lambda i: (jnp.maximum(i - 8, 0), 0)),
                   pl.BlockSpec((BM, H2), lambda i: (jnp.maximum(i - 8, 0), 0)),
                   pl.BlockSpec((1, 1), lambda i: (0, 0))],
        out_shape=[jax.ShapeDtypeStruct((N, H2), jnp.float32),
                   jax.ShapeDtypeStruct((N, H2), jnp.float32),
                   jax.ShapeDtypeStruct((N, H2), jnp.bfloat16),
                   jax.ShapeDtypeStruct((1, 1), jnp.float32)],
        scratch_shapes=[pltpu.VMEM((N, N), jnp.bfloat16),
                        pltpu.VMEM((N, H1), jnp.float32),
                        pltpu.VMEM((N, 2 * H2), jnp.bfloat16),
                        pltpu.VMEM((BM, H2), jnp.float32)],
        compiler_params=pltpu.CompilerParams(
            vmem_limit_bytes=100 * 1024 * 1024),
    )(adj, xw1, W23, eps)
    return z_mean, z, zb, klsum


def _decode(z):
    s1, cge = pl.pallas_call(
        _decode_kernel,
        grid=(N // DB, N // DB),
        in_specs=[pl.BlockSpec((DB, H2), lambda i, j: (i, 0)),
                  pl.BlockSpec((N, H2), lambda i, j: (0, 0))],
        out_specs=[pl.BlockSpec((1, 1), lambda i, j: (0, 0)),
                   pl.BlockSpec((1, 1), lambda i, j: (0, 0))],
        out_shape=[jax.ShapeDtypeStruct((1, 1), jnp.float32),
                   jax.ShapeDtypeStruct((1, 1), jnp.float32)],
        scratch_shapes=[pltpu.VMEM((1, DB), jnp.float32),
                        pltpu.VMEM((1, DB), jnp.float32)],
    )(z, z)
    return s1[0, 0], cge[0, 0]


_E_PER_TILE = NNZ // 32
_CHUNK = 256
_GROUPS = _CHUNK // 16


def _sc_sparse_body(z_hbm, ii_hbm, jj_hbm, vv_hbm, out_hbm,
                    ii_v, jj_v, vv_v, zi_v, zj_v, q_v, o_v, sem0, sem1):
    c = lax.axis_index("c")
    s = lax.axis_index("s")
    wid = s * 2 + c
    base = wid * _E_PER_TILE
    pltpu.sync_copy(ii_hbm.at[pl.ds(base, _E_PER_TILE)], ii_v)
    pltpu.sync_copy(jj_hbm.at[pl.ds(base, _E_PER_TILE)], jj_v)
    pltpu.sync_copy(vv_hbm.at[pl.ds(base, _E_PER_TILE)], vv_v)

    lanes = lax.iota(jnp.int32, 16)
    zero16 = jnp.zeros((16,), jnp.float32)

    def chunk_body(ck, carry):
        acc_s2, acc_corr = carry
        off = ck * _CHUNK
        pltpu.async_copy(z_hbm.at[ii_v.at[pl.ds(off, _CHUNK)]], zi_v, sem0).wait()
        pltpu.async_copy(z_hbm.at[jj_v.at[pl.ds(off, _CHUNK)]], zj_v, sem1).wait()

        def row_body(r, _):
            p0 = zi_v[r, pl.ds(0, 16)] * zj_v[r, pl.ds(0, 16)]
            p1 = zi_v[r, pl.ds(16, 16)] * zj_v[r, pl.ds(16, 16)]
            p2 = zi_v[r, pl.ds(32, 16)] * zj_v[r, pl.ds(32, 16)]
            p3 = zi_v[r, pl.ds(48, 16)] * zj_v[r, pl.ds(48, 16)]
            q_v[pl.ds(r * 16, 16)] = (p0 + p1) + (p2 + p3)
            return 0

        lax.fori_loop(0, _CHUNK, row_body, 0)

        def group_body(g, carry2):
            a_s2, a_corr = carry2
            flat0 = (g * 16 + lanes) * 16
            x = zero16
            for l in range(16):
                x = x + plsc.load_gather(q_v, [flat0 + l])
            vals = vv_v[pl.ds(off + g * 16, 16)]
            a_s2 = a_s2 + vals * x
            a_corr = a_corr + jnp.where(x >= 0.0, 1.0, -1.0)
            return (a_s2, a_corr)

        return lax.fori_loop(0, _GROUPS, group_body, (acc_s2, acc_corr))

    acc_s2, acc_corr = lax.fori_loop(
        0, _E_PER_TILE // _CHUNK, chunk_body, (zero16, zero16))
    o_v[0, :] = acc_s2
    o_v[1, :] = acc_corr
    pltpu.sync_copy(o_v, out_hbm.at[wid])


def _sc_sparse(z, idx_i, idx_j, values):
    mesh = plsc.VectorSubcoreMesh(core_axis_name="c", subcore_axis_name="s")
    run = pl.kernel(
        _sc_sparse_body,
        out_type=jax.ShapeDtypeStruct((32, 2, 16), jnp.float32),
        mesh=mesh,
        compiler_params=pltpu.CompilerParams(needs_layout_passes=False,
                                             use_tc_tiling_on_sc=False),
        scratch_types=[
            pltpu.VMEM((_E_PER_TILE,), jnp.int32),
            pltpu.VMEM((_E_PER_TILE,), jnp.int32),
            pltpu.VMEM((_E_PER_TILE,), jnp.float32),
            pltpu.VMEM((_CHUNK, H2), jnp.float32),
            pltpu.VMEM((_CHUNK, H2), jnp.float32),
            pltpu.VMEM((_CHUNK * 16,), jnp.float32),
            pltpu.VMEM((2, 16), jnp.float32),
            pltpu.SemaphoreType.DMA,
            pltpu.SemaphoreType.DMA,
        ],
    )
    return run(z, idx_i, idx_j, values)


def kernel(features, adj, labels_indices, labels_values, W1, W2, W3):
    W23 = jnp.concatenate([W2, W3], axis=1)
    eps = jax.random.normal(jax.random.key(42), (N, H2), dtype=jnp.float32)

    z_mean, z, zb, klsum = _dense_chain(features, adj, eps, W1, W23)
    idx_i = labels_indices[:, 0].astype(jnp.int32)
    idx_j = labels_indices[:, 1].astype(jnp.int32)
    sc_out = _sc_sparse(z, idx_i, idx_j, labels_values)
    s1, cge = _decode(zb)
    s2 = jnp.sum(sc_out[:, 0, :])
    corr = jnp.sum(sc_out[:, 1, :])

    n2 = jnp.float32(N * N)
    cost_pre = (s1 - s2) / n2
    kl = 0.5 * klsum[0, 0] / n2
    cost = cost_pre - kl
    accuracy = ((n2 - cge) + corr) / n2
    return (cost, accuracy, z_mean, cost_pre)


# speedup vs baseline: 2.8698x; 1.0008x over previous
import functools

import jax
import jax.numpy as jnp
from jax import lax
from jax.experimental import pallas as pl
from jax.experimental.pallas import tpu as pltpu
from jax.experimental.pallas import tpu_sc as plsc

N = 4096
F_IN = 256
H1 = 128
H2 = 64
NNZ = 65536

BM = 512
DB = 512


def _matmul_kernel(x_ref, w_ref, o_ref):
    o_ref[...] = lax.dot_general(
        x_ref[...], w_ref[...], (((1,), (0,)), ((), ())),
        preferred_element_type=jnp.float32)
def _gcn_kernel(a_ref, xw1_ref, w23_ref, eps_ref,
                zm_ref, z_ref, zb_ref, kl_ref,
                adjb, h1s, hw23b, klacc):
    i = pl.program_id(0)

    @pl.when(i < 8)
    def _():
        blk = a_ref[...]
        acc = lax.dot_general(
            blk, xw1_ref[...], (((1,), (0,)), ((), ())),
            preferred_element_type=jnp.float32)
        h1s[pl.ds(i * BM, BM), :] = jnp.maximum(acc, 0.0)
        adjb[pl.ds(i * BM, BM), :] = blk.astype(jnp.bfloat16)

    @pl.when(i == 8)
    def _():
        hw = lax.dot_general(
            h1s[...], w23_ref[...], (((1,), (0,)), ((), ())),
            preferred_element_type=jnp.float32)
        hw23b[...] = hw.astype(jnp.bfloat16)

    @pl.when(i >= 8)
    def _():
        r = jnp.maximum(i - 8, 0)
        zc = lax.dot_general(
            adjb[pl.ds(r * BM, BM), :], hw23b[...], (((1,), (0,)), ((), ())),
            preferred_element_type=jnp.float32)
        zm = zc[:, :H2]
        zs = zc[:, H2:]
        ez = jnp.exp(zs)
        zv = zm + eps_ref[...] * ez
        zm_ref[...] = zm
        z_ref[...] = zv
        zb_ref[...] = zv.astype(jnp.bfloat16)
        term = 1.0 + 2.0 * zs - zm * zm - ez * ez

        @pl.when(i == 8)
        def _():
            klacc[...] = term

        @pl.when(i > 8)
        def _():
            klacc[...] += term

        @pl.when(i == 15)
        def _():
            kl_ref[...] = jnp.sum(klacc[...], keepdims=True)


def _decode_kernel(zi_ref, zf_ref, s1_ref, cge_ref, accs, accc):
    i = pl.program_id(0)
    j = pl.program_id(1)
    first = jnp.logical_and(i == 0, j == 0)
    zj = zf_ref[pl.ds(j * DB, DB), :]
    x = lax.dot_general(
        zi_ref[...], zj, (((1,), (1,)), ((), ())),
        preferred_element_type=jnp.float32)
    sp = jnp.maximum(x, 0.0) + jnp.log1p(jnp.exp(-jnp.abs(x)))
    ge = (x >= 0.0).astype(jnp.float32)
    sp_part = jnp.sum(sp, axis=0, keepdims=True)
    ge_part = jnp.sum(ge, axis=0, keepdims=True)

    @pl.when(first)
    def _():
        accs[...] = sp_part
        accc[...] = ge_part

    @pl.when(jnp.logical_not(first))
    def _():
        accs[...] += sp_part
        accc[...] += ge_part

    @pl.when(jnp.logical_and(i == pl.num_programs(0) - 1,
                             j == pl.num_programs(1) - 1))
    def _():
        s1_ref[...] = jnp.sum(accs[...], keepdims=True)
        cge_ref[...] = jnp.sum(accc[...], keepdims=True)


def _dense_chain(features, adj, eps, W1, W23):
    xw1 = pl.pallas_call(
        _matmul_kernel,
        grid=(N // BM,),
        in_specs=[pl.BlockSpec((BM, F_IN), lambda i: (i, 0)),
                  pl.BlockSpec((F_IN, H1), lambda i: (0, 0))],
        out_specs=pl.BlockSpec((BM, H1), lambda i: (i, 0)),
        out_shape=jax.ShapeDtypeStruct((N, H1), jnp.float32),
    )(features, W1)

    z_mean, z, zb, klsum = pl.pallas_call(
        _gcn_kernel,
        grid=(16,),
        in_specs=[pl.BlockSpec((BM, N), lambda i: (jnp.minimum(i, 7), 0)),
                  pl.BlockSpec((N, H1), lambda i: (0, 0)),
                  pl.BlockSpec((H1, 2 * H2), lambda i: (0, 0)),
                  pl.BlockSpec((BM, H2), lambda i: (jnp.maximum(i - 8, 0), 0))],
        out_specs=[pl.BlockSpec((BM, H2), lambda i: (jnp.maximum(i - 8, 0), 0)),
                   pl.BlockSpec((BM, H2), lambda i: (jnp.maximum(i - 8, 0), 0)),
                   pl.BlockSpec((BM, H2), lambda i: (jnp.maximum(i - 8, 0), 0)),
                   pl.BlockSpec((1, 1), lambda i: (0, 0))],
        out_shape=[jax.ShapeDtypeStruct((N, H2), jnp.float32),
                   jax.ShapeDtypeStruct((N, H2), jnp.float32),
                   jax.ShapeDtypeStruct((N, H2), jnp.bfloat16),
                   jax.ShapeDtypeStruct((1, 1), jnp.float32)],
        scratch_shapes=[pltpu.VMEM((N, N), jnp.bfloat16),
                        pltpu.VMEM((N, H1), jnp.float32),
                        pltpu.VMEM((N, 2 * H2), jnp.bfloat16),
                        pltpu.VMEM((BM, H2), jnp.float32)],
        compiler_params=pltpu.CompilerParams(
            vmem_limit_bytes=100 * 1024 * 1024),
    )(adj, xw1, W23, eps)
    return z_mean, z, zb, klsum


def _decode(z):
    s1, cge = pl.pallas_call(
        _decode_kernel,
        grid=(N // DB, N // DB),
        in_specs=[pl.BlockSpec((DB, H2), lambda i, j: (i, 0)),
                  pl.BlockSpec((N, H2), lambda i, j: (0, 0))],
        out_specs=[pl.BlockSpec((1, 1), lambda i, j: (0, 0)),
                   pl.BlockSpec((1, 1), lambda i, j: (0, 0))],
        out_shape=[jax.ShapeDtypeStruct((1, 1), jnp.float32),
                   jax.ShapeDtypeStruct((1, 1), jnp.float32)],
        scratch_shapes=[pltpu.VMEM((1, DB), jnp.float32),
                        pltpu.VMEM((1, DB), jnp.float32)],
    )(z, z)
    return s1[0, 0], cge[0, 0]


_E_PER_TILE = NNZ // 32
_CHUNK = 256
_GROUPS = _CHUNK // 16


def _sc_sparse_body(z_hbm, ii_hbm, jj_hbm, vv_hbm, out_hbm,
                    ii_v, jj_v, vv_v, zi0_v, zj0_v, zi1_v, zj1_v, q_v, o_v,
                    sem_i0, sem_j0, sem_i1, sem_j1):
    c = lax.axis_index("c")
    s = lax.axis_index("s")
    wid = s * 2 + c
    base = wid * _E_PER_TILE
    pltpu.sync_copy(ii_hbm.at[pl.ds(base, _E_PER_TILE)], ii_v)
    pltpu.sync_copy(jj_hbm.at[pl.ds(base, _E_PER_TILE)], jj_v)
    pltpu.sync_copy(vv_hbm.at[pl.ds(base, _E_PER_TILE)], vv_v)

    lanes = lax.iota(jnp.int32, 16)
    zero16 = jnp.zeros((16,), jnp.float32)
    n_chunks = _E_PER_TILE // _CHUNK
    bufs = ((zi0_v, zj0_v, sem_i0, sem_j0), (zi1_v, zj1_v, sem_i1, sem_j1))

    def issue(ck, buf):
        zi_b, zj_b, s_i, s_j = buf
        off = ck * _CHUNK
        cp_i = pltpu.async_copy(z_hbm.at[ii_v.at[pl.ds(off, _CHUNK)]], zi_b, s_i)
        cp_j = pltpu.async_copy(z_hbm.at[jj_v.at[pl.ds(off, _CHUNK)]], zj_b, s_j)
        return cp_i, cp_j

    acc_s2 = zero16
    acc_corr = zero16
    pend = issue(0, bufs[0])
    for ck in range(n_chunks):
        zi_b, zj_b, s_i, s_j = bufs[ck % 2]
        pend[0].wait()
        pend[1].wait()
        if ck + 1 < n_chunks:
            pend = issue(ck + 1, bufs[(ck + 1) % 2])

        def row_body(r4, _, zi_b=zi_b, zj_b=zj_b):
            for u in range(4):
                r = r4 * 4 + u
                p0 = zi_b[r, pl.ds(0, 16)] * zj_b[r, pl.ds(0, 16)]
                p1 = zi_b[r, pl.ds(16, 16)] * zj_b[r, pl.ds(16, 16)]
                p2 = zi_b[r, pl.ds(32, 16)] * zj_b[r, pl.ds(32, 16)]
                p3 = zi_b[r, pl.ds(48, 16)] * zj_b[r, pl.ds(48, 16)]
                q_v[pl.ds(r * 16, 16)] = (p0 + p1) + (p2 + p3)
            return 0

        lax.fori_loop(0, _CHUNK // 4, row_body, 0)

        off = ck * _CHUNK

        def group_body(g, carry2, off=off):
            a_s2, a_corr = carry2
            flat0 = (g * 16 + lanes) * 16
            x = zero16
            for l in range(16):
                x = x + plsc.load_gather(q_v, [flat0 + l])
            vals = vv_v[pl.ds(off + g * 16, 16)]
            a_s2 = a_s2 + vals * x
            a_corr = a_corr + jnp.where(x >= 0.0, 1.0, -1.0)
            return (a_s2, a_corr)

        acc_s2, acc_corr = lax.fori_loop(
            0, _GROUPS, group_body, (acc_s2, acc_corr))

    o_v[0, :] = acc_s2
    o_v[1, :] = acc_corr
    pltpu.sync_copy(o_v, out_hbm.at[wid])


def _sc_sparse(z, idx_i, idx_j, values):
    mesh = plsc.VectorSubcoreMesh(core_axis_name="c", subcore_axis_name="s")
    run = pl.kernel(
        _sc_sparse_body,
        out_type=jax.ShapeDtypeStruct((32, 2, 16), jnp.float32),
        mesh=mesh,
        compiler_params=pltpu.CompilerParams(needs_layout_passes=False,
                                             use_tc_tiling_on_sc=False),
        scratch_types=[
            pltpu.VMEM((_E_PER_TILE,), jnp.int32),
            pltpu.VMEM((_E_PER_TILE,), jnp.int32),
            pltpu.VMEM((_E_PER_TILE,), jnp.float32),
            pltpu.VMEM((_CHUNK, H2), jnp.float32),
            pltpu.VMEM((_CHUNK, H2), jnp.float32),
            pltpu.VMEM((_CHUNK, H2), jnp.float32),
            pltpu.VMEM((_CHUNK, H2), jnp.float32),
            pltpu.VMEM((_CHUNK * 16,), jnp.float32),
            pltpu.VMEM((2, 16), jnp.float32),
            pltpu.SemaphoreType.DMA,
            pltpu.SemaphoreType.DMA,
            pltpu.SemaphoreType.DMA,
            pltpu.SemaphoreType.DMA,
        ],
    )
    return run(z, idx_i, idx_j, values)


def kernel(features, adj, labels_indices, labels_values, W1, W2, W3):
    W23 = jnp.concatenate([W2, W3], axis=1)
    eps = jax.random.normal(jax.random.key(42), (N, H2), dtype=jnp.float32)

    z_mean, z, zb, klsum = _dense_chain(features, adj, eps, W1, W23)
    idx_i = labels_indices[:, 0].astype(jnp.int32)
    idx_j = labels_indices[:, 1].astype(jnp.int32)
    sc_out = _sc_sparse(z, idx_i, idx_j, labels_values)
    s1, cge = _decode(zb)
    s2 = jnp.sum(sc_out[:, 0, :])
    corr = jnp.sum(sc_out[:, 1, :])

    n2 = jnp.float32(N * N)
    cost_pre = (s1 - s2) / n2
    kl = 0.5 * klsum[0, 0] / n2
    cost = cost_pre - kl
    accuracy = ((n2 - cge) + corr) / n2
    return (cost, accuracy, z_mean, cost_pre)


# speedup vs baseline: 3.3716x; 1.1748x over previous
import functools

import jax
import jax.numpy as jnp
from jax import lax
from jax.experimental import pallas as pl
from jax.experimental.pallas import tpu as pltpu
from jax.experimental.pallas import tpu_sc as plsc

N = 4096
F_IN = 256
H1 = 128
H2 = 64
NNZ = 65536

BM = 512
DB = 512


def _matmul_bf16_kernel(x_ref, w_ref, o_ref):
    o_ref[...] = lax.dot_general(
        x_ref[...], w_ref[...], (((1,), (0,)), ((), ())),
        preferred_element_type=jnp.float32).astype(jnp.bfloat16)
def _gcn_kernel(a_ref, xw1_ref, w23_ref, eps_ref,
                zm_ref, z_ref, zb_ref, kl_ref,
                adjb, h1s, hw23b, klacc):
    i = pl.program_id(0)

    @pl.when(i < 8)
    def _():
        blkb = a_ref[...].astype(jnp.bfloat16)
        acc = lax.dot_general(
            blkb, xw1_ref[...], (((1,), (0,)), ((), ())),
            preferred_element_type=jnp.float32)
        h1s[pl.ds(i * BM, BM), :] = jnp.maximum(acc, 0.0)
        adjb[pl.ds(i * BM, BM), :] = blkb

    @pl.when(i == 8)
    def _():
        hw = lax.dot_general(
            h1s[...], w23_ref[...], (((1,), (0,)), ((), ())),
            preferred_element_type=jnp.float32)
        hw23b[...] = hw.astype(jnp.bfloat16)

    @pl.when(i >= 8)
    def _():
        r = jnp.maximum(i - 8, 0)
        zc = lax.dot_general(
            adjb[pl.ds(r * BM, BM), :], hw23b[...], (((1,), (0,)), ((), ())),
            preferred_element_type=jnp.float32)
        zm = zc[:, :H2]
        zs = zc[:, H2:]
        ez = jnp.exp(zs)
        zv = zm + eps_ref[...] * ez
        zm_ref[...] = zm
        z_ref[...] = zv
        zb_ref[...] = zv.astype(jnp.bfloat16)
        term = 1.0 + 2.0 * zs - zm * zm - ez * ez

        @pl.when(i == 8)
        def _():
            klacc[...] = term

        @pl.when(i > 8)
        def _():
            klacc[...] += term

        @pl.when(i == 15)
        def _():
            kl_ref[...] = jnp.sum(klacc[...], keepdims=True)


def _decode_kernel(zi_ref, zf_ref, s1_ref, cge_ref, accs, accc):
    i = pl.program_id(0)
    j = pl.program_id(1)

    @pl.when(jnp.logical_and(i == 0, j == 0))
    def _():
        accs[...] = jnp.zeros_like(accs)
        accc[...] = jnp.zeros_like(accc)

    @pl.when(j >= i)
    def _():
        zj = zf_ref[pl.ds(j * DB, DB), :]
        x = lax.dot_general(
            zi_ref[...], zj, (((1,), (1,)), ((), ())),
            preferred_element_type=jnp.float32)
        sp = jnp.maximum(x, 0.0) + jnp.log1p(jnp.exp(-jnp.abs(x)))
        ge = (x >= 0.0).astype(jnp.float32)
        w = jnp.where(i == j, 1.0, 2.0).astype(jnp.float32)
        accs[...] += w * jnp.sum(sp, axis=0, keepdims=True)
        accc[...] += w * jnp.sum(ge, axis=0, keepdims=True)

    @pl.when(jnp.logical_and(i == pl.num_programs(0) - 1,
                             j == pl.num_programs(1) - 1))
    def _():
        s1_ref[...] = jnp.sum(accs[...], keepdims=True)
        cge_ref[...] = jnp.sum(accc[...], keepdims=True)


def _dense_chain(features, adj, eps, W1, W23):
    xw1 = pl.pallas_call(
        _matmul_bf16_kernel,
        grid=(N // BM,),
        in_specs=[pl.BlockSpec((BM, F_IN), lambda i: (i, 0)),
                  pl.BlockSpec((F_IN, H1), lambda i: (0, 0))],
        out_specs=pl.BlockSpec((BM, H1), lambda i: (i, 0)),
        out_shape=jax.ShapeDtypeStruct((N, H1), jnp.bfloat16),
    )(features, W1)

    z_mean, z, zb, klsum = pl.pallas_call(
        _gcn_kernel,
        grid=(16,),
        in_specs=[pl.BlockSpec((BM, N), lambda i: (jnp.minimum(i, 7), 0)),
                  pl.BlockSpec((N, H1), lambda i: (0, 0)),
                  pl.BlockSpec((H1, 2 * H2), lambda i: (0, 0)),
                  pl.BlockSpec((BM, H2), lambda i: (jnp.maximum(i - 8, 0), 0))],
        out_specs=[pl.BlockSpec((BM, H2), lambda i: (jnp.maximum(i - 8, 0), 0)),
                   pl.BlockSpec((BM, H2), lambda i: (jnp.maximum(i - 8, 0), 0)),
                   pl.BlockSpec((BM, H2), lambda i: (jnp.maximum(i - 8, 0), 0)),
                   pl.BlockSpec((1, 1), lambda i: (0, 0))],
        out_shape=[jax.ShapeDtypeStruct((N, H2), jnp.float32),
                   jax.ShapeDtypeStruct((N, H2), jnp.float32),
                   jax.ShapeDtypeStruct((N, H2), jnp.bfloat16),
                   jax.ShapeDtypeStruct((1, 1), jnp.float32)],
        scratch_shapes=[pltpu.VMEM((N, N), jnp.bfloat16),
                        pltpu.VMEM((N, H1), jnp.float32),
                        pltpu.VMEM((N, 2 * H2), jnp.bfloat16),
                        pltpu.VMEM((BM, H2), jnp.float32)],
        compiler_params=pltpu.CompilerParams(
            vmem_limit_bytes=100 * 1024 * 1024),
    )(adj, xw1, W23, eps)
    return z_mean, z, zb, klsum


def _decode(z):
    s1, cge = pl.pallas_call(
        _decode_kernel,
        grid=(N // DB, N // DB),
        in_specs=[pl.BlockSpec((DB, H2), lambda i, j: (i, 0)),
                  pl.BlockSpec((N, H2), lambda i, j: (0, 0))],
        out_specs=[pl.BlockSpec((1, 1), lambda i, j: (0, 0)),
                   pl.BlockSpec((1, 1), lambda i, j: (0, 0))],
        out_shape=[jax.ShapeDtypeStruct((1, 1), jnp.float32),
                   jax.ShapeDtypeStruct((1, 1), jnp.float32)],
        scratch_shapes=[pltpu.VMEM((1, DB), jnp.float32),
                        pltpu.VMEM((1, DB), jnp.float32)],
    )(z, z)
    return s1[0, 0], cge[0, 0]


_E_PER_TILE = NNZ // 32
_CHUNK = 256
_GROUPS = _CHUNK // 16


def _sc_sparse_body(z_hbm, ii_hbm, jj_hbm, vv_hbm, out_hbm,
                    ii_v, jj_v, vv_v, zi0_v, zj0_v, zi1_v, zj1_v, q_v, o_v,
                    sem_i0, sem_j0, sem_i1, sem_j1):
    c = lax.axis_index("c")
    s = lax.axis_index("s")
    wid = s * 2 + c
    base = wid * _E_PER_TILE
    pltpu.sync_copy(ii_hbm.at[pl.ds(base, _E_PER_TILE)], ii_v)
    pltpu.sync_copy(jj_hbm.at[pl.ds(base, _E_PER_TILE)], jj_v)
    pltpu.sync_copy(vv_hbm.at[pl.ds(base, _E_PER_TILE)], vv_v)

    lanes = lax.iota(jnp.int32, 16)
    zero16 = jnp.zeros((16,), jnp.float32)
    n_chunks = _E_PER_TILE // _CHUNK
    bufs = ((zi0_v, zj0_v, sem_i0, sem_j0), (zi1_v, zj1_v, sem_i1, sem_j1))

    def issue(ck, buf):
        zi_b, zj_b, s_i, s_j = buf
        off = ck * _CHUNK
        cp_i = pltpu.async_copy(z_hbm.at[ii_v.at[pl.ds(off, _CHUNK)]], zi_b, s_i)
        cp_j = pltpu.async_copy(z_hbm.at[jj_v.at[pl.ds(off, _CHUNK)]], zj_b, s_j)
        return cp_i, cp_j

    acc_s2 = zero16
    acc_corr = zero16
    pend = issue(0, bufs[0])
    for ck in range(n_chunks):
        zi_b, zj_b, s_i, s_j = bufs[ck % 2]
        pend[0].wait()
        pend[1].wait()
        if ck + 1 < n_chunks:
            pend = issue(ck + 1, bufs[(ck + 1) % 2])

        def row_body(r4, _, zi_b=zi_b, zj_b=zj_b):
            for u in range(4):
                r = r4 * 4 + u
                p0 = zi_b[r, pl.ds(0, 16)] * zj_b[r, pl.ds(0, 16)]
                p1 = zi_b[r, pl.ds(16, 16)] * zj_b[r, pl.ds(16, 16)]
                p2 = zi_b[r, pl.ds(32, 16)] * zj_b[r, pl.ds(32, 16)]
                p3 = zi_b[r, pl.ds(48, 16)] * zj_b[r, pl.ds(48, 16)]
                q_v[pl.ds(r * 16, 16)] = (p0 + p1) + (p2 + p3)
            return 0

        lax.fori_loop(0, _CHUNK // 4, row_body, 0)

        off = ck * _CHUNK

        def group_body(g, carry2, off=off):
            a_s2, a_corr = carry2
            flat0 = (g * 16 + lanes) * 16
            x = zero16
            for l in range(16):
                x = x + plsc.load_gather(q_v, [flat0 + l])
            vals = vv_v[pl.ds(off + g * 16, 16)]
            a_s2 = a_s2 + vals * x
            a_corr = a_corr + jnp.where(x >= 0.0, 1.0, -1.0)
            return (a_s2, a_corr)

        acc_s2, acc_corr = lax.fori_loop(
            0, _GROUPS, group_body, (acc_s2, acc_corr))

    o_v[0, :] = acc_s2
    o_v[1, :] = acc_corr
    pltpu.sync_copy(o_v, out_hbm.at[wid])


def _sc_sparse(z, idx_i, idx_j, values):
    mesh = plsc.VectorSubcoreMesh(core_axis_name="c", subcore_axis_name="s")
    run = pl.kernel(
        _sc_sparse_body,
        out_type=jax.ShapeDtypeStruct((32, 2, 16), jnp.float32),
        mesh=mesh,
        compiler_params=pltpu.CompilerParams(needs_layout_passes=False,
                                             use_tc_tiling_on_sc=False),
        scratch_types=[
            pltpu.VMEM((_E_PER_TILE,), jnp.int32),
            pltpu.VMEM((_E_PER_TILE,), jnp.int32),
            pltpu.VMEM((_E_PER_TILE,), jnp.float32),
            pltpu.VMEM((_CHUNK, H2), jnp.float32),
            pltpu.VMEM((_CHUNK, H2), jnp.float32),
            pltpu.VMEM((_CHUNK, H2), jnp.float32),
            pltpu.VMEM((_CHUNK, H2), jnp.float32),
            pltpu.VMEM((_CHUNK * 16,), jnp.float32),
            pltpu.VMEM((2, 16), jnp.float32),
            pltpu.SemaphoreType.DMA,
            pltpu.SemaphoreType.DMA,
            pltpu.SemaphoreType.DMA,
            pltpu.SemaphoreType.DMA,
        ],
    )
    return run(z, idx_i, idx_j, values)


def kernel(features, adj, labels_indices, labels_values, W1, W2, W3):
    W23 = jnp.concatenate([W2, W3], axis=1)
    eps = jax.random.normal(jax.random.key(42), (N, H2), dtype=jnp.float32)

    z_mean, z, zb, klsum = _dense_chain(features, adj, eps, W1, W23)
    idx_i = labels_indices[:, 0].astype(jnp.int32)
    idx_j = labels_indices[:, 1].astype(jnp.int32)
    sc_out = _sc_sparse(z, idx_i, idx_j, labels_values)
    s1, cge = _decode(zb)
    s2 = jnp.sum(sc_out[:, 0, :])
    corr = jnp.sum(sc_out[:, 1, :])

    n2 = jnp.float32(N * N)
    cost_pre = (s1 - s2) / n2
    kl = 0.5 * klsum[0, 0] / n2
    cost = cost_pre - kl
    accuracy = ((n2 - cge) + corr) / n2
    return (cost, accuracy, z_mean, cost_pre)
